# Initial kernel scaffold; baseline (speedup 1.0000x reference)
import jax
import jax.numpy as jnp
from jax.experimental import pallas as pl


def kernel(data, org_edge_index, emb_table, lin_W, att_i, att_j, att_em_i, att_em_j, gnn_bias, bn1_gamma, bn1_beta, bn2_gamma, bn2_beta, W_out, b_out):
    raise NotImplementedError("write your pallas kernel here")


# speedup vs baseline: 49.8538x; 49.8538x over previous
import functools

import jax
import jax.numpy as jnp
from jax import lax
from jax.experimental import pallas as pl
from jax.experimental.pallas import tpu as pltpu
from jax.experimental.pallas import tpu_sc as plsc

_B, _N, _W, _D = 128, 1000, 64, 64
_K = 20
_BN = _B * _N
_E = _B * _N * _K
_NEG = 0.2

_NC, _NS = 2, 16
_NWORK = _NC * _NS
_CH = 128
_NCHUNK = _E // _CH // _NWORK
_XW = 2 * _D


def _xlin_body(d_ref, w_ref, e_ref, xl_ref, xe_ref):
    xl = jnp.dot(d_ref[0], w_ref[...], preferred_element_type=jnp.float32)
    xl_ref[0] = xl
    xe_ref[0] = jnp.concatenate([xl, e_ref[...]], axis=-1)


def _xlin(data3, lin_W, emb):
    return pl.pallas_call(
        _xlin_body,
        grid=(_B,),
        in_specs=[pl.BlockSpec((1, _N, _W), lambda b: (b, 0, 0)),
                  pl.BlockSpec((_W, _D), lambda b: (0, 0)),
                  pl.BlockSpec((_N, _D), lambda b: (0, 0))],
        out_specs=[pl.BlockSpec((1, _N, _D), lambda b: (b, 0, 0)),
                   pl.BlockSpec((1, _N, _XW), lambda b: (b, 0, 0))],
        out_shape=[jax.ShapeDtypeStruct((_B, _N, _D), jnp.float32),
                   jax.ShapeDtypeStruct((_B, _N, _XW), jnp.float32)],
    )(data3, lin_W, emb)


_G3 = 3
_GR = _G3 * _CH
_NGRP = _NCHUNK // _G3


def _sc_gather_body(xe, srcg2, xrow_o,
                    idx_a, idx_b, rows_a, rows_b, gs_a, gs_b, ws_a, ws_b):
    wid = lax.axis_index("s") * _NC + lax.axis_index("c")
    cbase = wid * _NCHUNK

    def do_group(g, idx_v, rows_v, gsem, wsem, first):
        crow = cbase + g * _G3
        for j in range(_G3):
            pltpu.async_copy(srcg2.at[crow + j], idx_v.at[j], gsem)
        pltpu.make_async_copy(srcg2.at[pl.ds(0, _G3)], idx_v, gsem).wait()

        @pl.when(jnp.logical_not(first))
        def _():
            pltpu.make_async_copy(xrow_o.at[pl.ds(0, _GR)], rows_v, wsem).wait()

        for j in range(_G3):
            pltpu.async_copy(xe.at[idx_v.at[j]],
                             rows_v.at[pl.ds(j * _CH, _CH)], gsem)
        pltpu.make_async_copy(xrow_o.at[pl.ds(0, _GR)], rows_v, gsem).wait()
        pltpu.async_copy(rows_v, xrow_o.at[pl.ds(crow * _CH, _GR)], wsem)

    def dbl(gg, carry):
        do_group(gg * 2, idx_a, rows_a, gs_a, ws_a, gg == 0)
        do_group(gg * 2 + 1, idx_b, rows_b, gs_b, ws_b, gg == 0)
        return carry

    lax.fori_loop(0, _NGRP // 2, dbl, 0)

    crow = cbase + _NGRP * _G3
    pltpu.sync_copy(srcg2.at[crow], idx_a.at[0])
    pltpu.make_async_copy(xrow_o.at[pl.ds(0, _GR)], rows_a, ws_a).wait()
    pltpu.async_copy(xe.at[idx_a.at[0]], rows_a.at[pl.ds(0, _CH)], gs_a)
    pltpu.make_async_copy(xrow_o.at[pl.ds(0, _CH)],
                          rows_a.at[pl.ds(0, _CH)], gs_a).wait()
    pltpu.sync_copy(rows_a.at[pl.ds(0, _CH)],
                    xrow_o.at[pl.ds(crow * _CH, _CH)])
    pltpu.make_async_copy(xrow_o.at[pl.ds(0, _GR)], rows_b, ws_b).wait()


def _sc_gather(xe_flat, srcg2):
    kern = functools.partial(
        pl.kernel,
        out_type=jax.ShapeDtypeStruct((_E, _XW), jnp.float32),
        mesh=plsc.VectorSubcoreMesh(core_axis_name="c", subcore_axis_name="s"),
        scratch_types=[pltpu.VMEM((_G3, _CH), jnp.int32),
                       pltpu.VMEM((_G3, _CH), jnp.int32),
                       pltpu.VMEM((_GR, _XW), jnp.float32),
                       pltpu.VMEM((_GR, _XW), jnp.float32),
                       pltpu.SemaphoreType.DMA,
                       pltpu.SemaphoreType.DMA,
                       pltpu.SemaphoreType.DMA,
                       pltpu.SemaphoreType.DMA],
    )(_sc_gather_body)
    return kern(xe_flat, srcg2)


def _att_body(xrow, xlin, emb, mask, ai, aj, aei, aej, gb,
              att_o, atts_o, agg_o, s_o, ss_o):
    xr = xrow[0][:, :, :_D]
    ejr = xrow[0][:, :, _D:]
    xl = xlin[0]
    em = emb[...]
    m = mask[...]

    ci = jnp.sum(em * aei[...], axis=-1, keepdims=True)
    cj_self = jnp.sum(em * aej[...], axis=-1, keepdims=True)
    si = jnp.sum(xl * ai[...], axis=-1, keepdims=True) + ci
    sj_self = jnp.sum(xl * aj[...], axis=-1, keepdims=True) + cj_self
    sjg = (jnp.sum(xr * aj[...][None], axis=-1)
           + jnp.sum(ejr * aej[...][None], axis=-1))

    alpha = si + sjg
    alpha = jnp.where(alpha >= 0, alpha, _NEG * alpha)
    aself = si + sj_self
    aself = jnp.where(aself >= 0, aself, _NEG * aself)

    alpha_m = jnp.where(m > 0, -1e9, alpha)
    mx = jnp.maximum(jnp.max(alpha_m, axis=-1, keepdims=True), aself)
    ex = jnp.where(m > 0, 0.0, jnp.exp(alpha_m - mx))
    exs = jnp.exp(aself - mx)
    denom = jnp.sum(ex, axis=-1, keepdims=True) + exs
    att = ex / denom
    atts = exs / denom

    agg = jnp.sum(xr * att[:, :, None], axis=1) + atts * xl + gb[...]

    att_o[0] = att
    atts_o[0] = atts
    agg_o[0] = agg

    @pl.when(pl.program_id(0) == 0)
    def _():
        s_o[...] = jnp.zeros_like(s_o)
        ss_o[...] = jnp.zeros_like(ss_o)

    s_o[...] += jnp.sum(agg.reshape(_N // 8, 8, _D), axis=0)
    ss_o[...] += jnp.sum((agg * agg).reshape(_N // 8, 8, _D), axis=0)


def _attention(xrow4, xlin3, emb, maskf, ai, aj, aei, aej, gb):
    return pl.pallas_call(
        _att_body,
        grid=(_B,),
        in_specs=[
            pl.BlockSpec((1, _N, _K, _XW), lambda b: (b, 0, 0, 0)),
            pl.BlockSpec((1, _N, _D), lambda b: (b, 0, 0)),
            pl.BlockSpec((_N, _D), lambda b: (0, 0)),
            pl.BlockSpec((_N, _K), lambda b: (0, 0)),
            pl.BlockSpec((1, _D), lambda b: (0, 0)),
            pl.BlockSpec((1, _D), lambda b: (0, 0)),
            pl.BlockSpec((1, _D), lambda b: (0, 0)),
            pl.BlockSpec((1, _D), lambda b: (0, 0)),
            pl.BlockSpec((1, _D), lambda b: (0, 0)),
        ],
        out_specs=[
            pl.BlockSpec((1, _N, _K), lambda b: (b, 0, 0)),
            pl.BlockSpec((1, _N, 1), lambda b: (b, 0, 0)),
            pl.BlockSpec((1, _N, _D), lambda b: (b, 0, 0)),
            pl.BlockSpec((8, _D), lambda b: (0, 0)),
            pl.BlockSpec((8, _D), lambda b: (0, 0)),
        ],
        out_shape=[
            jax.ShapeDtypeStruct((_B, _N, _K), jnp.float32),
            jax.ShapeDtypeStruct((_B, _N, 1), jnp.float32),
            jax.ShapeDtypeStruct((_B, _N, _D), jnp.float32),
            jax.ShapeDtypeStruct((8, _D), jnp.float32),
            jax.ShapeDtypeStruct((8, _D), jnp.float32),
        ],
    )(xrow4, xlin3, emb, maskf, ai, aj, aei, aej, gb)


def _bn1_body(agg, emb, a1, b1, h_o, s_o, ss_o):
    h = agg[0] * a1[...] + b1[...]
    h = jnp.maximum(h, 0.0) * emb[...]
    h_o[0] = h

    @pl.when(pl.program_id(0) == 0)
    def _():
        s_o[...] = jnp.zeros_like(s_o)
        ss_o[...] = jnp.zeros_like(ss_o)

    s_o[...] += jnp.sum(h.reshape(_N // 8, 8, _D), axis=0)
    ss_o[...] += jnp.sum((h * h).reshape(_N // 8, 8, _D), axis=0)


def _bn1(agg3, emb, a1, b1):
    return pl.pallas_call(
        _bn1_body,
        grid=(_B,),
        in_specs=[
            pl.BlockSpec((1, _N, _D), lambda b: (b, 0, 0)),
            pl.BlockSpec((_N, _D), lambda b: (0, 0)),
            pl.BlockSpec((1, _D), lambda b: (0, 0)),
            pl.BlockSpec((1, _D), lambda b: (0, 0)),
        ],
        out_specs=[
            pl.BlockSpec((1, _N, _D), lambda b: (b, 0, 0)),
            pl.BlockSpec((8, _D), lambda b: (0, 0)),
            pl.BlockSpec((8, _D), lambda b: (0, 0)),
        ],
        out_shape=[
            jax.ShapeDtypeStruct((_B, _N, _D), jnp.float32),
            jax.ShapeDtypeStruct((8, _D), jnp.float32),
            jax.ShapeDtypeStruct((8, _D), jnp.float32),
        ],
    )(agg3, emb, a1, b1)


def _head_body(h, a2, b2, wo, bo, p_o):
    xg = h[0] * a2[...] + b2[...]
    xg = jnp.maximum(xg, 0.0)
    p_o[0] = jnp.dot(xg, wo[...], preferred_element_type=jnp.float32) + bo[...]


def _head(h3, a2, b2, wo, bo):
    return pl.pallas_call(
        _head_body,
        grid=(_B,),
        in_specs=[
            pl.BlockSpec((1, _N, _D), lambda b: (b, 0, 0)),
            pl.BlockSpec((1, _D), lambda b: (0, 0)),
            pl.BlockSpec((1, _D), lambda b: (0, 0)),
            pl.BlockSpec((_D, 1), lambda b: (0, 0)),
            pl.BlockSpec((1, 1), lambda b: (0, 0)),
        ],
        out_specs=pl.BlockSpec((1, _N, 1), lambda b: (b, 0, 0)),
        out_shape=jax.ShapeDtypeStruct((_B, _N, 1), jnp.float32),
    )(h3, a2, b2, wo, bo)


def kernel(data, org_edge_index, emb_table, lin_W, att_i, att_j, att_em_i,
           att_em_j, gnn_bias, bn1_gamma, bn1_beta, bn2_gamma, bn2_beta,
           W_out, b_out):
    weights = lax.stop_gradient(emb_table)
    cos = weights @ weights.T
    norms = jnp.linalg.norm(weights, axis=-1)
    cos = cos / (norms[:, None] * norms[None, :])
    _, topk_idx = lax.top_k(cos, _K)

    gated_i = jnp.repeat(jnp.arange(_N), _K)
    gated_j = topk_idx.reshape(-1)
    offs = (jnp.arange(_B) * _N)[:, None]
    src = (gated_j[None, :] + offs).reshape(-1)
    dst = (gated_i[None, :] + offs).reshape(-1)
    loop = jnp.arange(_BN)
    src_all = jnp.concatenate([src, loop])
    dst_all = jnp.concatenate([dst, loop])
    edge_index_out = jnp.stack([src_all, dst_all])
    maskf = (topk_idx == jnp.arange(_N)[:, None]).astype(jnp.float32)

    xlin3, xe = _xlin(data.reshape(_B, _N, _W), lin_W, emb_table)

    xrow = _sc_gather(xe.reshape(_BN, _XW),
                      src.astype(jnp.int32).reshape(_E // _CH, _CH))

    att_g, att_s, agg, s1, ss1 = _attention(
        xrow.reshape(_B, _N, _K, _XW), xlin3, emb_table, maskf,
        att_i.reshape(1, _D), att_j.reshape(1, _D),
        att_em_i.reshape(1, _D), att_em_j.reshape(1, _D),
        gnn_bias.reshape(1, _D))

    att = jnp.concatenate([att_g.reshape(-1), att_s.reshape(-1)])

    cnt = jnp.float32(_BN)
    mu1 = s1.sum(0) / cnt
    var1 = ss1.sum(0) / cnt - mu1 * mu1
    a1 = bn1_gamma / jnp.sqrt(var1 + 1e-5)
    b1 = bn1_beta - mu1 * a1

    h, s2, ss2 = _bn1(agg, emb_table, a1.reshape(1, _D), b1.reshape(1, _D))

    mu2 = s2.sum(0) / cnt
    var2 = ss2.sum(0) / cnt - mu2 * mu2
    a2 = bn2_gamma / jnp.sqrt(var2 + 1e-5)
    b2 = bn2_beta - mu2 * a2

    pred = _head(h, a2.reshape(1, _D), b2.reshape(1, _D),
                 W_out, b_out.reshape(1, 1)).reshape(-1, _N)

    return (pred, att, edge_index_out, topk_idx, weights)


# speedup vs baseline: 51.2622x; 1.0283x over previous
import functools

import jax
import jax.numpy as jnp
from jax import lax
from jax.experimental import pallas as pl
from jax.experimental.pallas import tpu as pltpu
from jax.experimental.pallas import tpu_sc as plsc

_B, _N, _W, _D = 128, 1000, 64, 64
_K = 20
_BN = _B * _N
_E = _B * _N * _K
_NEG = 0.2

_NC, _NS = 2, 16
_NWORK = _NC * _NS
_CH = 128
_NCHUNK = _E // _CH // _NWORK
_XW = 2 * _D


def _topk_body(emb_ref, embT_ref, idx_o, cos_v):
    em = emb_ref[...]
    emT = embT_ref[...]
    nc = jnp.sqrt(jnp.sum(em * em, axis=-1, keepdims=True))
    nr = jnp.sqrt(jnp.sum(emT * emT, axis=0, keepdims=True))
    cos = jnp.dot(em, emT, preferred_element_type=jnp.float32)
    cos_v[...] = cos / (nc * nr)
    lane = lax.broadcasted_iota(jnp.int32, (_N, _N), 1)
    cols = []
    for _ in range(_K):
        c = cos_v[...]
        mx = jnp.max(c, axis=1, keepdims=True)
        idx = jnp.min(jnp.where(c == mx, lane, _N), axis=1, keepdims=True)
        cols.append(idx)
        cos_v[...] = jnp.where(lane == idx, -3.4e38, c)
    idx_o[...] = jnp.concatenate(cols, axis=1)


def _topk(emb, embT):
    return pl.pallas_call(
        _topk_body,
        out_shape=jax.ShapeDtypeStruct((_N, _K), jnp.int32),
        scratch_shapes=[pltpu.VMEM((_N, _N), jnp.float32)],
    )(emb, embT)


def _xlin_body(d_ref, w_ref, e_ref, xl_ref, xe_ref):
    xl = jnp.dot(d_ref[0], w_ref[...], preferred_element_type=jnp.float32)
    xl_ref[0] = xl
    xe_ref[0] = jnp.concatenate([xl, e_ref[...]], axis=-1)


def _xlin(data3, lin_W, emb):
    return pl.pallas_call(
        _xlin_body,
        grid=(_B,),
        in_specs=[pl.BlockSpec((1, _N, _W), lambda b: (b, 0, 0)),
                  pl.BlockSpec((_W, _D), lambda b: (0, 0)),
                  pl.BlockSpec((_N, _D), lambda b: (0, 0))],
        out_specs=[pl.BlockSpec((1, _N, _D), lambda b: (b, 0, 0)),
                   pl.BlockSpec((1, _N, _XW), lambda b: (b, 0, 0))],
        out_shape=[jax.ShapeDtypeStruct((_B, _N, _D), jnp.float32),
                   jax.ShapeDtypeStruct((_B, _N, _XW), jnp.float32)],
    )(data3, lin_W, emb)


_G3 = 3
_GR = _G3 * _CH
_NGRP = _NCHUNK // _G3


def _sc_gather_body(xe, srcg2, xrow_o,
                    idx_a, idx_b, rows_a, rows_b, gs_a, gs_b, ws_a, ws_b):
    wid = lax.axis_index("s") * _NC + lax.axis_index("c")
    cbase = wid * _NCHUNK

    def do_group(g, idx_v, rows_v, gsem, wsem, first):
        crow = cbase + g * _G3
        for j in range(_G3):
            pltpu.async_copy(srcg2.at[crow + j], idx_v.at[j], gsem)
        pltpu.make_async_copy(srcg2.at[pl.ds(0, _G3)], idx_v, gsem).wait()

        @pl.when(jnp.logical_not(first))
        def _():
            pltpu.make_async_copy(xrow_o.at[pl.ds(0, _GR)], rows_v, wsem).wait()

        for j in range(_G3):
            pltpu.async_copy(xe.at[idx_v.at[j]],
                             rows_v.at[pl.ds(j * _CH, _CH)], gsem)
        pltpu.make_async_copy(xrow_o.at[pl.ds(0, _GR)], rows_v, gsem).wait()
        pltpu.async_copy(rows_v, xrow_o.at[pl.ds(crow * _CH, _GR)], wsem)

    def dbl(gg, carry):
        do_group(gg * 2, idx_a, rows_a, gs_a, ws_a, gg == 0)
        do_group(gg * 2 + 1, idx_b, rows_b, gs_b, ws_b, gg == 0)
        return carry

    lax.fori_loop(0, _NGRP // 2, dbl, 0)

    crow = cbase + _NGRP * _G3
    pltpu.sync_copy(srcg2.at[crow], idx_a.at[0])
    pltpu.make_async_copy(xrow_o.at[pl.ds(0, _GR)], rows_a, ws_a).wait()
    pltpu.async_copy(xe.at[idx_a.at[0]], rows_a.at[pl.ds(0, _CH)], gs_a)
    pltpu.make_async_copy(xrow_o.at[pl.ds(0, _CH)],
                          rows_a.at[pl.ds(0, _CH)], gs_a).wait()
    pltpu.sync_copy(rows_a.at[pl.ds(0, _CH)],
                    xrow_o.at[pl.ds(crow * _CH, _CH)])
    pltpu.make_async_copy(xrow_o.at[pl.ds(0, _GR)], rows_b, ws_b).wait()


def _sc_gather(xe_flat, srcg2):
    kern = functools.partial(
        pl.kernel,
        out_type=jax.ShapeDtypeStruct((_E, _XW), jnp.float32),
        mesh=plsc.VectorSubcoreMesh(core_axis_name="c", subcore_axis_name="s"),
        scratch_types=[pltpu.VMEM((_G3, _CH), jnp.int32),
                       pltpu.VMEM((_G3, _CH), jnp.int32),
                       pltpu.VMEM((_GR, _XW), jnp.float32),
                       pltpu.VMEM((_GR, _XW), jnp.float32),
                       pltpu.SemaphoreType.DMA,
                       pltpu.SemaphoreType.DMA,
                       pltpu.SemaphoreType.DMA,
                       pltpu.SemaphoreType.DMA],
    )(_sc_gather_body)
    return kern(xe_flat, srcg2)


def _att_body(xrow, xlin, emb, mask, ai, aj, aei, aej, gb,
              att_o, atts_o, agg_o, s_o, ss_o):
    xr = xrow[0][:, :, :_D]
    ejr = xrow[0][:, :, _D:]
    xl = xlin[0]
    em = emb[...]
    m = mask[...]

    ci = jnp.sum(em * aei[...], axis=-1, keepdims=True)
    cj_self = jnp.sum(em * aej[...], axis=-1, keepdims=True)
    si = jnp.sum(xl * ai[...], axis=-1, keepdims=True) + ci
    sj_self = jnp.sum(xl * aj[...], axis=-1, keepdims=True) + cj_self
    sjg = (jnp.sum(xr * aj[...][None], axis=-1)
           + jnp.sum(ejr * aej[...][None], axis=-1))

    alpha = si + sjg
    alpha = jnp.where(alpha >= 0, alpha, _NEG * alpha)
    aself = si + sj_self
    aself = jnp.where(aself >= 0, aself, _NEG * aself)

    alpha_m = jnp.where(m > 0, -1e9, alpha)
    mx = jnp.maximum(jnp.max(alpha_m, axis=-1, keepdims=True), aself)
    ex = jnp.where(m > 0, 0.0, jnp.exp(alpha_m - mx))
    exs = jnp.exp(aself - mx)
    denom = jnp.sum(ex, axis=-1, keepdims=True) + exs
    att = ex / denom
    atts = exs / denom

    agg = jnp.sum(xr * att[:, :, None], axis=1) + atts * xl + gb[...]

    att_o[0] = att
    atts_o[0] = atts
    agg_o[0] = agg

    @pl.when(pl.program_id(0) == 0)
    def _():
        s_o[...] = jnp.zeros_like(s_o)
        ss_o[...] = jnp.zeros_like(ss_o)

    s_o[...] += jnp.sum(agg.reshape(_N // 8, 8, _D), axis=0)
    ss_o[...] += jnp.sum((agg * agg).reshape(_N // 8, 8, _D), axis=0)


def _attention(xrow4, xlin3, emb, maskf, ai, aj, aei, aej, gb):
    return pl.pallas_call(
        _att_body,
        grid=(_B,),
        in_specs=[
            pl.BlockSpec((1, _N, _K, _XW), lambda b: (b, 0, 0, 0)),
            pl.BlockSpec((1, _N, _D), lambda b: (b, 0, 0)),
            pl.BlockSpec((_N, _D), lambda b: (0, 0)),
            pl.BlockSpec((_N, _K), lambda b: (0, 0)),
            pl.BlockSpec((1, _D), lambda b: (0, 0)),
            pl.BlockSpec((1, _D), lambda b: (0, 0)),
            pl.BlockSpec((1, _D), lambda b: (0, 0)),
            pl.BlockSpec((1, _D), lambda b: (0, 0)),
            pl.BlockSpec((1, _D), lambda b: (0, 0)),
        ],
        out_specs=[
            pl.BlockSpec((1, _N, _K), lambda b: (b, 0, 0)),
            pl.BlockSpec((1, _N, 1), lambda b: (b, 0, 0)),
            pl.BlockSpec((1, _N, _D), lambda b: (b, 0, 0)),
            pl.BlockSpec((8, _D), lambda b: (0, 0)),
            pl.BlockSpec((8, _D), lambda b: (0, 0)),
        ],
        out_shape=[
            jax.ShapeDtypeStruct((_B, _N, _K), jnp.float32),
            jax.ShapeDtypeStruct((_B, _N, 1), jnp.float32),
            jax.ShapeDtypeStruct((_B, _N, _D), jnp.float32),
            jax.ShapeDtypeStruct((8, _D), jnp.float32),
            jax.ShapeDtypeStruct((8, _D), jnp.float32),
        ],
    )(xrow4, xlin3, emb, maskf, ai, aj, aei, aej, gb)


def _bn1_body(agg, emb, a1, b1, h_o, s_o, ss_o):
    h = agg[0] * a1[...] + b1[...]
    h = jnp.maximum(h, 0.0) * emb[...]
    h_o[0] = h

    @pl.when(pl.program_id(0) == 0)
    def _():
        s_o[...] = jnp.zeros_like(s_o)
        ss_o[...] = jnp.zeros_like(ss_o)

    s_o[...] += jnp.sum(h.reshape(_N // 8, 8, _D), axis=0)
    ss_o[...] += jnp.sum((h * h).reshape(_N // 8, 8, _D), axis=0)


def _bn1(agg3, emb, a1, b1):
    return pl.pallas_call(
        _bn1_body,
        grid=(_B,),
        in_specs=[
            pl.BlockSpec((1, _N, _D), lambda b: (b, 0, 0)),
            pl.BlockSpec((_N, _D), lambda b: (0, 0)),
            pl.BlockSpec((1, _D), lambda b: (0, 0)),
            pl.BlockSpec((1, _D), lambda b: (0, 0)),
        ],
        out_specs=[
            pl.BlockSpec((1, _N, _D), lambda b: (b, 0, 0)),
            pl.BlockSpec((8, _D), lambda b: (0, 0)),
            pl.BlockSpec((8, _D), lambda b: (0, 0)),
        ],
        out_shape=[
            jax.ShapeDtypeStruct((_B, _N, _D), jnp.float32),
            jax.ShapeDtypeStruct((8, _D), jnp.float32),
            jax.ShapeDtypeStruct((8, _D), jnp.float32),
        ],
    )(agg3, emb, a1, b1)


def _head_body(h, a2, b2, wo, bo, p_o):
    xg = h[0] * a2[...] + b2[...]
    xg = jnp.maximum(xg, 0.0)
    p_o[0] = jnp.dot(xg, wo[...], preferred_element_type=jnp.float32) + bo[...]


def _head(h3, a2, b2, wo, bo):
    return pl.pallas_call(
        _head_body,
        grid=(_B,),
        in_specs=[
            pl.BlockSpec((1, _N, _D), lambda b: (b, 0, 0)),
            pl.BlockSpec((1, _D), lambda b: (0, 0)),
            pl.BlockSpec((1, _D), lambda b: (0, 0)),
            pl.BlockSpec((_D, 1), lambda b: (0, 0)),
            pl.BlockSpec((1, 1), lambda b: (0, 0)),
        ],
        out_specs=pl.BlockSpec((1, _N, 1), lambda b: (b, 0, 0)),
        out_shape=jax.ShapeDtypeStruct((_B, _N, 1), jnp.float32),
    )(h3, a2, b2, wo, bo)


def kernel(data, org_edge_index, emb_table, lin_W, att_i, att_j, att_em_i,
           att_em_j, gnn_bias, bn1_gamma, bn1_beta, bn2_gamma, bn2_beta,
           W_out, b_out):
    weights = lax.stop_gradient(emb_table)
    topk_idx = _topk(weights, weights.T)

    gated_i = jnp.repeat(jnp.arange(_N), _K)
    gated_j = topk_idx.reshape(-1)
    offs = (jnp.arange(_B) * _N)[:, None]
    src = (gated_j[None, :] + offs).reshape(-1)
    dst = (gated_i[None, :] + offs).reshape(-1)
    loop = jnp.arange(_BN)
    src_all = jnp.concatenate([src, loop])
    dst_all = jnp.concatenate([dst, loop])
    edge_index_out = jnp.stack([src_all, dst_all])
    maskf = (topk_idx == jnp.arange(_N)[:, None]).astype(jnp.float32)

    xlin3, xe = _xlin(data.reshape(_B, _N, _W), lin_W, emb_table)

    xrow = _sc_gather(xe.reshape(_BN, _XW),
                      src.astype(jnp.int32).reshape(_E // _CH, _CH))

    att_g, att_s, agg, s1, ss1 = _attention(
        xrow.reshape(_B, _N, _K, _XW), xlin3, emb_table, maskf,
        att_i.reshape(1, _D), att_j.reshape(1, _D),
        att_em_i.reshape(1, _D), att_em_j.reshape(1, _D),
        gnn_bias.reshape(1, _D))

    att = jnp.concatenate([att_g.reshape(-1), att_s.reshape(-1)])

    cnt = jnp.float32(_BN)
    mu1 = s1.sum(0) / cnt
    var1 = ss1.sum(0) / cnt - mu1 * mu1
    a1 = bn1_gamma / jnp.sqrt(var1 + 1e-5)
    b1 = bn1_beta - mu1 * a1

    h, s2, ss2 = _bn1(agg, emb_table, a1.reshape(1, _D), b1.reshape(1, _D))

    mu2 = s2.sum(0) / cnt
    var2 = ss2.sum(0) / cnt - mu2 * mu2
    a2 = bn2_gamma / jnp.sqrt(var2 + 1e-5)
    b2 = bn2_beta - mu2 * a2

    pred = _head(h, a2.reshape(1, _D), b2.reshape(1, _D),
                 W_out, b_out.reshape(1, 1)).reshape(-1, _N)

    return (pred, att, edge_index_out, topk_idx, weights)


# speedup vs baseline: 61.8987x; 1.2075x over previous
import functools

import jax
import jax.numpy as jnp
from jax import lax
from jax.experimental import pallas as pl
from jax.experimental.pallas import tpu as pltpu
from jax.experimental.pallas import tpu_sc as plsc

_B, _N, _W, _D = 128, 1000, 64, 64
_K = 20
_BN = _B * _N
_E = _B * _N * _K
_NEG = 0.2

_NC, _NS = 2, 16
_NWORK = _NC * _NS
_CH = 128
_NCHUNK = _E // _CH // _NWORK
_XW = 2 * _D


def _topk_body(emb_ref, embT_ref, idx_o, cos_v):
    em = emb_ref[...]
    emT = embT_ref[...]
    nc = jnp.sqrt(jnp.sum(em * em, axis=-1, keepdims=True))
    nr = jnp.sqrt(jnp.sum(emT * emT, axis=0, keepdims=True))
    cos = jnp.dot(em, emT, preferred_element_type=jnp.float32)
    cos_v[...] = cos / (nc * nr)
    lane = lax.broadcasted_iota(jnp.int32, (_N, _N), 1)
    cols = []
    for _ in range(_K):
        c = cos_v[...]
        mx = jnp.max(c, axis=1, keepdims=True)
        idx = jnp.min(jnp.where(c == mx, lane, _N), axis=1, keepdims=True)
        cols.append(idx)
        cos_v[...] = jnp.where(lane == idx, -3.4e38, c)
    idx_o[...] = jnp.concatenate(cols, axis=1)


def _topk(emb, embT):
    return pl.pallas_call(
        _topk_body,
        out_shape=jax.ShapeDtypeStruct((_N, _K), jnp.int32),
        scratch_shapes=[pltpu.VMEM((_N, _N), jnp.float32)],
    )(emb, embT)


def _xlin_body(d_ref, w_ref, e_ref, xl_ref, xe_ref):
    xl = jnp.dot(d_ref[0], w_ref[...], preferred_element_type=jnp.float32)
    xl_ref[0] = xl
    xe_ref[0] = jnp.concatenate([xl, e_ref[...]], axis=-1)


def _xlin(data3, lin_W, emb):
    return pl.pallas_call(
        _xlin_body,
        grid=(_B,),
        in_specs=[pl.BlockSpec((1, _N, _W), lambda b: (b, 0, 0)),
                  pl.BlockSpec((_W, _D), lambda b: (0, 0)),
                  pl.BlockSpec((_N, _D), lambda b: (0, 0))],
        out_specs=[pl.BlockSpec((1, _N, _D), lambda b: (b, 0, 0)),
                   pl.BlockSpec((1, _N, _XW), lambda b: (b, 0, 0))],
        out_shape=[jax.ShapeDtypeStruct((_B, _N, _D), jnp.float32),
                   jax.ShapeDtypeStruct((_B, _N, _XW), jnp.float32)],
    )(data3, lin_W, emb)


_G3 = 3
_GR = _G3 * _CH
_NGRP = _NCHUNK // _G3


def _sc_gather_body(xe, srcg2, xrow_o,
                    idx_a, idx_b, rows_a, rows_b, gs_a, gs_b, ws_a, ws_b):
    wid = lax.axis_index("s") * _NC + lax.axis_index("c")
    cbase = wid * _NCHUNK

    def do_group(g, idx_v, rows_v, gsem, wsem, first):
        crow = cbase + g * _G3
        for j in range(_G3):
            pltpu.async_copy(srcg2.at[crow + j], idx_v.at[j], gsem)
        pltpu.make_async_copy(srcg2.at[pl.ds(0, _G3)], idx_v, gsem).wait()

        @pl.when(jnp.logical_not(first))
        def _():
            pltpu.make_async_copy(xrow_o.at[pl.ds(0, _GR)], rows_v, wsem).wait()

        for j in range(_G3):
            pltpu.async_copy(xe.at[idx_v.at[j]],
                             rows_v.at[pl.ds(j * _CH, _CH)], gsem)
        pltpu.make_async_copy(xrow_o.at[pl.ds(0, _GR)], rows_v, gsem).wait()
        pltpu.async_copy(rows_v, xrow_o.at[pl.ds(crow * _CH, _GR)], wsem)

    def dbl(gg, carry):
        do_group(gg * 2, idx_a, rows_a, gs_a, ws_a, gg == 0)
        do_group(gg * 2 + 1, idx_b, rows_b, gs_b, ws_b, gg == 0)
        return carry

    lax.fori_loop(0, _NGRP // 2, dbl, 0)

    crow = cbase + _NGRP * _G3
    pltpu.sync_copy(srcg2.at[crow], idx_a.at[0])
    pltpu.make_async_copy(xrow_o.at[pl.ds(0, _GR)], rows_a, ws_a).wait()
    pltpu.async_copy(xe.at[idx_a.at[0]], rows_a.at[pl.ds(0, _CH)], gs_a)
    pltpu.make_async_copy(xrow_o.at[pl.ds(0, _CH)],
                          rows_a.at[pl.ds(0, _CH)], gs_a).wait()
    pltpu.sync_copy(rows_a.at[pl.ds(0, _CH)],
                    xrow_o.at[pl.ds(crow * _CH, _CH)])
    pltpu.make_async_copy(xrow_o.at[pl.ds(0, _GR)], rows_b, ws_b).wait()


def _sc_gather(xe_flat, srcg2):
    kern = functools.partial(
        pl.kernel,
        out_type=jax.ShapeDtypeStruct((_E, _XW), jnp.float32),
        mesh=plsc.VectorSubcoreMesh(core_axis_name="c", subcore_axis_name="s"),
        scratch_types=[pltpu.VMEM((_G3, _CH), jnp.int32),
                       pltpu.VMEM((_G3, _CH), jnp.int32),
                       pltpu.VMEM((_GR, _XW), jnp.float32),
                       pltpu.VMEM((_GR, _XW), jnp.float32),
                       pltpu.SemaphoreType.DMA,
                       pltpu.SemaphoreType.DMA,
                       pltpu.SemaphoreType.DMA,
                       pltpu.SemaphoreType.DMA],
    )(_sc_gather_body)
    return kern(xe_flat, srcg2)


def _att_body(xrow, xlin, emb, mask, ai, catj, aj, aei, aej, gb,
              att_o, atts_o, agg_o, s_o, ss_o):
    xr = xrow[0]
    xl = xlin[0]
    em = emb[...]
    m = mask[...]

    ci = jnp.sum(em * aei[...], axis=-1, keepdims=True)
    cj_self = jnp.sum(em * aej[...], axis=-1, keepdims=True)
    si = jnp.sum(xl * ai[...], axis=-1, keepdims=True) + ci
    sj_self = jnp.sum(xl * aj[...], axis=-1, keepdims=True) + cj_self
    sjg = jnp.sum(xr * catj[...][None], axis=-1)

    alpha = si + sjg
    alpha = jnp.where(alpha >= 0, alpha, _NEG * alpha)
    aself = si + sj_self
    aself = jnp.where(aself >= 0, aself, _NEG * aself)

    alpha_m = jnp.where(m > 0, -1e9, alpha)
    mx = jnp.maximum(jnp.max(alpha_m, axis=-1, keepdims=True), aself)
    ex = jnp.where(m > 0, 0.0, jnp.exp(alpha_m - mx))
    exs = jnp.exp(aself - mx)
    denom = jnp.sum(ex, axis=-1, keepdims=True) + exs
    att = ex / denom
    atts = exs / denom

    agg_f = jnp.sum(xr * att[:, :, None], axis=1)
    agg = agg_f[:, :_D] + atts * xl + gb[...]

    att_o[0] = att
    atts_o[0] = atts
    agg_o[0] = agg

    @pl.when(pl.program_id(0) == 0)
    def _():
        s_o[...] = jnp.zeros_like(s_o)
        ss_o[...] = jnp.zeros_like(ss_o)

    s_o[...] += jnp.sum(agg.reshape(_N // 8, 8, _D), axis=0)
    ss_o[...] += jnp.sum((agg * agg).reshape(_N // 8, 8, _D), axis=0)


def _attention(xrow4, xlin3, emb, maskf, ai, catj, aj, aei, aej, gb):
    return pl.pallas_call(
        _att_body,
        grid=(_B,),
        in_specs=[
            pl.BlockSpec((1, _N, _K, _XW), lambda b: (b, 0, 0, 0)),
            pl.BlockSpec((1, _N, _D), lambda b: (b, 0, 0)),
            pl.BlockSpec((_N, _D), lambda b: (0, 0)),
            pl.BlockSpec((_N, _K), lambda b: (0, 0)),
            pl.BlockSpec((1, _D), lambda b: (0, 0)),
            pl.BlockSpec((1, _XW), lambda b: (0, 0)),
            pl.BlockSpec((1, _D), lambda b: (0, 0)),
            pl.BlockSpec((1, _D), lambda b: (0, 0)),
            pl.BlockSpec((1, _D), lambda b: (0, 0)),
            pl.BlockSpec((1, _D), lambda b: (0, 0)),
        ],
        out_specs=[
            pl.BlockSpec((1, _N, _K), lambda b: (b, 0, 0)),
            pl.BlockSpec((1, _N, 1), lambda b: (b, 0, 0)),
            pl.BlockSpec((1, _N, _D), lambda b: (b, 0, 0)),
            pl.BlockSpec((8, _D), lambda b: (0, 0)),
            pl.BlockSpec((8, _D), lambda b: (0, 0)),
        ],
        out_shape=[
            jax.ShapeDtypeStruct((_B, _N, _K), jnp.float32),
            jax.ShapeDtypeStruct((_B, _N, 1), jnp.float32),
            jax.ShapeDtypeStruct((_B, _N, _D), jnp.float32),
            jax.ShapeDtypeStruct((8, _D), jnp.float32),
            jax.ShapeDtypeStruct((8, _D), jnp.float32),
        ],
    )(xrow4, xlin3, emb, maskf, ai, catj, aj, aei, aej, gb)


def _bn1_body(agg, emb, a1, b1, h_o, s_o, ss_o):
    h = agg[0] * a1[...] + b1[...]
    h = jnp.maximum(h, 0.0) * emb[...]
    h_o[0] = h

    @pl.when(pl.program_id(0) == 0)
    def _():
        s_o[...] = jnp.zeros_like(s_o)
        ss_o[...] = jnp.zeros_like(ss_o)

    s_o[...] += jnp.sum(h.reshape(_N // 8, 8, _D), axis=0)
    ss_o[...] += jnp.sum((h * h).reshape(_N // 8, 8, _D), axis=0)


def _bn1(agg3, emb, a1, b1):
    return pl.pallas_call(
        _bn1_body,
        grid=(_B,),
        in_specs=[
            pl.BlockSpec((1, _N, _D), lambda b: (b, 0, 0)),
            pl.BlockSpec((_N, _D), lambda b: (0, 0)),
            pl.BlockSpec((1, _D), lambda b: (0, 0)),
            pl.BlockSpec((1, _D), lambda b: (0, 0)),
        ],
        out_specs=[
            pl.BlockSpec((1, _N, _D), lambda b: (b, 0, 0)),
            pl.BlockSpec((8, _D), lambda b: (0, 0)),
            pl.BlockSpec((8, _D), lambda b: (0, 0)),
        ],
        out_shape=[
            jax.ShapeDtypeStruct((_B, _N, _D), jnp.float32),
            jax.ShapeDtypeStruct((8, _D), jnp.float32),
            jax.ShapeDtypeStruct((8, _D), jnp.float32),
        ],
    )(agg3, emb, a1, b1)


def _head_body(h, a2, b2, wo, bo, p_o):
    xg = h[0] * a2[...] + b2[...]
    xg = jnp.maximum(xg, 0.0)
    p_o[0] = jnp.dot(xg, wo[...], preferred_element_type=jnp.float32) + bo[...]


def _head(h3, a2, b2, wo, bo):
    return pl.pallas_call(
        _head_body,
        grid=(_B,),
        in_specs=[
            pl.BlockSpec((1, _N, _D), lambda b: (b, 0, 0)),
            pl.BlockSpec((1, _D), lambda b: (0, 0)),
            pl.BlockSpec((1, _D), lambda b: (0, 0)),
            pl.BlockSpec((_D, 1), lambda b: (0, 0)),
            pl.BlockSpec((1, 1), lambda b: (0, 0)),
        ],
        out_specs=pl.BlockSpec((1, _N, 1), lambda b: (b, 0, 0)),
        out_shape=jax.ShapeDtypeStruct((_B, _N, 1), jnp.float32),
    )(h3, a2, b2, wo, bo)


def kernel(data, org_edge_index, emb_table, lin_W, att_i, att_j, att_em_i,
           att_em_j, gnn_bias, bn1_gamma, bn1_beta, bn2_gamma, bn2_beta,
           W_out, b_out):
    weights = lax.stop_gradient(emb_table)
    topk_idx = _topk(weights, weights.T)

    gated_i = jnp.repeat(jnp.arange(_N), _K)
    gated_j = topk_idx.reshape(-1)
    offs = (jnp.arange(_B) * _N)[:, None]
    src = (gated_j[None, :] + offs).reshape(-1)
    dst = (gated_i[None, :] + offs).reshape(-1)
    loop = jnp.arange(_BN)
    src_all = jnp.concatenate([src, loop])
    dst_all = jnp.concatenate([dst, loop])
    edge_index_out = jnp.stack([src_all, dst_all])
    maskf = (topk_idx == jnp.arange(_N)[:, None]).astype(jnp.float32)

    xlin3, xe = _xlin(data.reshape(_B, _N, _W), lin_W, emb_table)

    xrow = _sc_gather(xe.reshape(_BN, _XW),
                      src.astype(jnp.int32).reshape(_E // _CH, _CH))

    att_g, att_s, agg, s1, ss1 = _attention(
        xrow.reshape(_B, _N, _K, _XW), xlin3, emb_table, maskf,
        att_i.reshape(1, _D),
        jnp.concatenate([att_j, att_em_j]).reshape(1, _XW),
        att_j.reshape(1, _D), att_em_i.reshape(1, _D),
        att_em_j.reshape(1, _D), gnn_bias.reshape(1, _D))

    att = jnp.concatenate([att_g.reshape(-1), att_s.reshape(-1)])

    cnt = jnp.float32(_BN)
    mu1 = s1.sum(0) / cnt
    var1 = ss1.sum(0) / cnt - mu1 * mu1
    a1 = bn1_gamma / jnp.sqrt(var1 + 1e-5)
    b1 = bn1_beta - mu1 * a1

    h, s2, ss2 = _bn1(agg, emb_table, a1.reshape(1, _D), b1.reshape(1, _D))

    mu2 = s2.sum(0) / cnt
    var2 = ss2.sum(0) / cnt - mu2 * mu2
    a2 = bn2_gamma / jnp.sqrt(var2 + 1e-5)
    b2 = bn2_beta - mu2 * a2

    pred = _head(h, a2.reshape(1, _D), b2.reshape(1, _D),
                 W_out, b_out.reshape(1, 1)).reshape(-1, _N)

    return (pred, att, edge_index_out, topk_idx, weights)
